# baseline (speedup 1.0000x reference)
import jax
import jax.numpy as jnp
from jax.experimental import pallas as pl
from jax.experimental.pallas import tpu as pltpu

K_RATIO_ = 0.5
BLK = 2000


def _logits_body(h_ref, w_ref, out_ref):
    z = h_ref[...]
    w = w_ref[...]
    lg = jax.lax.dot_general(
        z.astype(jnp.bfloat16), w.astype(jnp.bfloat16),
        dimension_numbers=(((2,), (1,)), ((), ())),
        preferred_element_type=jnp.float32,
    )
    out_ref[...] = lg[None]


def _logits(h, W):
    B, N, D = h.shape
    grid = (N // BLK,)
    out = pl.pallas_call(
        _logits_body,
        grid=grid,
        in_specs=[
            pl.BlockSpec((B, BLK, D), lambda j: (0, j, 0)),
            pl.BlockSpec((1, D), lambda j: (0, 0)),
        ],
        out_specs=pl.BlockSpec((1, B, BLK, 1), lambda j: (j, 0, 0, 0)),
        out_shape=jax.ShapeDtypeStruct((N // BLK, B, BLK, 1), jnp.float32),
    )(h, W)
    return out[..., 0].transpose(1, 0, 2).reshape(B, N)


def kernel(h, W, b):
    B, N, D = h.shape
    n_keep = max(int(N * K_RATIO_), 1)
    s = jax.nn.sigmoid(_logits(h, W) + b)
    _, idx = jax.lax.top_k(s, n_keep)
    idx_full = jnp.broadcast_to(idx[:, :, None], (B, n_keep, D))
    hg = jnp.take_along_axis(h, idx_full, axis=1)
    sg = jnp.take_along_axis(s, idx, axis=1)
    return hg * sg[:, :, None]
